# Initial kernel scaffold; baseline (speedup 1.0000x reference)
import functools

import jax
import jax.numpy as jnp
from jax import lax
from jax.experimental import pallas as pl
from jax.experimental.pallas import tpu as pltpu
from jax.experimental.pallas import tpu_sc as plsc

B, S, H, NH = 4, 4096, 128, 8
HD = H // NH
N = B * S
E = 262144
LN_EPS = 1e-5
ALPHA = 0.1

NC, NS, L = 2, 16, 16
NW = NC * NS
C = 128
HH = H // NC
GPH = HH // L

_MESH = plsc.VectorSubcoreMesh(core_axis_name="c", subcore_axis_name="s")


def _f32(shape):
    return jax.ShapeDtypeStruct(shape, jnp.float32)


def _qkv_body(x_ref, wq_ref, wk_ref, wv_ref, bq_ref, bk_ref, bv_ref,
              q_ref, k_ref, v_ref):
    x = x_ref[...]
    dg = lambda a, w: lax.dot_general(a, w, (((1,), (1,)), ((), ())),
                                      preferred_element_type=jnp.float32)
    q_ref[...] = (dg(x, wq_ref[...]) + bq_ref[...]) * (1.0 / 4.0)
    k_ref[...] = dg(x, wk_ref[...]) + bk_ref[...]
    v_ref[...] = dg(x, wv_ref[...]) + bv_ref[...]


def _qkv(x, Wq, Wk, Wv, bq, bk, bv):
    blk = 1024
    row_spec = pl.BlockSpec((blk, H), lambda i: (i, 0))
    w_spec = pl.BlockSpec((H, H), lambda i: (0, 0))
    b_spec = pl.BlockSpec((1, H), lambda i: (0, 0))
    return pl.pallas_call(
        _qkv_body,
        grid=(N // blk,),
        in_specs=[row_spec, w_spec, w_spec, w_spec, b_spec, b_spec, b_spec],
        out_specs=[row_spec, row_spec, row_spec],
        out_shape=[_f32((N, H))] * 3,
    )(x, Wq, Wk, Wv, bq.reshape(1, H), bk.reshape(1, H), bv.reshape(1, H))


def _out_body(h_ref, x_ref, wo_ref, bo_ref, g_ref, b_ref, y_ref):
    h = h_ref[...]
    y = lax.dot_general(h, wo_ref[...], (((1,), (1,)), ((), ())),
                        preferred_element_type=jnp.float32)
    y = y + bo_ref[...] + x_ref[...]
    mu = jnp.mean(y, axis=-1, keepdims=True)
    var = jnp.mean((y - mu) ** 2, axis=-1, keepdims=True)
    y_ref[...] = (y - mu) * lax.rsqrt(var + LN_EPS) * g_ref[...] + b_ref[...]


def _out_proj(h, x, Wo, bo, g, b):
    blk = 1024
    row_spec = pl.BlockSpec((blk, H), lambda i: (i, 0))
    w_spec = pl.BlockSpec((H, H), lambda i: (0, 0))
    b_spec = pl.BlockSpec((1, H), lambda i: (0, 0))
    return pl.pallas_call(
        _out_body,
        grid=(N // blk,),
        in_specs=[row_spec, row_spec, w_spec, b_spec, b_spec, b_spec],
        out_specs=row_spec,
        out_shape=_f32((N, H)),
    )(h, x, Wo, bo.reshape(1, H), g.reshape(1, H), b.reshape(1, H))


EPT2 = E // NW
NCH2 = EPT2 // C
ROWS_PER_TILE = N // NS


@functools.partial(
    pl.kernel,
    out_type=[_f32((NH, E)), _f32((NC, N, 16))],
    mesh=_MESH,
    scratch_types=[
        pltpu.VMEM((C,), jnp.int32),
        pltpu.VMEM((C,), jnp.int32),
        pltpu.VMEM((C, H), jnp.float32),
        pltpu.VMEM((C, H), jnp.float32),
        pltpu.VMEM((NH, C), jnp.float32),
        pltpu.VMEM((C, 16), jnp.float32),
        pltpu.VMEM((N // NS, 16), jnp.float32),
        pltpu.VMEM_SHARED((N, 16), jnp.float32),
        pltpu.SemaphoreType.DMA,
        pltpu.SemaphoreType.DMA,
    ],
)
def _p2(k_hbm, q_hbm, src_hbm, dst_hbm, ex_out, parts_out,
        svec, dvec, krows, qrows, sct, pad, zbuf, denom, sem_a, sem_b):
    cid = lax.axis_index("c")
    sid = lax.axis_index("s")
    wid = cid * NS + sid
    base0 = wid * EPT2
    zero16 = jnp.zeros((L,), jnp.float32)

    def _zrow(i, _):
        zbuf[i, :] = zero16
        return 0
    lax.fori_loop(0, ROWS_PER_TILE, _zrow, 0)

    def _prow(i, _):
        pad[i, :] = zero16
        return 0
    lax.fori_loop(0, C, _prow, 0)

    pltpu.sync_copy(zbuf, denom.at[pl.ds(sid * ROWS_PER_TILE, ROWS_PER_TILE)])
    plsc.subcore_barrier()

    lanes = lax.iota(jnp.int32, L)

    def _chunk(c, _):
        base = base0 + c * C
        pltpu.sync_copy(src_hbm.at[pl.ds(base, C)], svec)
        pltpu.sync_copy(dst_hbm.at[pl.ds(base, C)], dvec)
        cp_k = pltpu.async_copy(k_hbm.at[svec], krows, sem_a)
        cp_q = pltpu.async_copy(q_hbm.at[dvec], qrows, sem_b)
        cp_k.wait()
        cp_q.wait()

        def _edge(e, _):
            for h in range(NH):
                kv = krows[e, pl.ds(h * HD, HD)]
                qv = qrows[e, pl.ds(h * HD, HD)]
                sct[h, e] = jnp.sum(kv * qv)
            return 0
        lax.fori_loop(0, C, _edge, 0)

        for h in range(NH):
            for grp in range(C // L):
                ex = jnp.exp(sct[h, pl.ds(grp * L, L)])
                sct[h, pl.ds(grp * L, L)] = ex
                plsc.store_scatter(
                    pad, [grp * L + lanes, jnp.full((L,), h, jnp.int32)], ex)

        pltpu.sync_copy(sct, ex_out.at[:, pl.ds(base, C)])
        pltpu.sync_copy(pad, denom.at[dvec], add=True)
        return 0

    lax.fori_loop(0, NCH2, _chunk, 0)
    plsc.subcore_barrier()

    r0 = sid * ROWS_PER_TILE
    pltpu.sync_copy(denom.at[pl.ds(r0, ROWS_PER_TILE)], zbuf)
    pltpu.sync_copy(zbuf, parts_out.at[cid, pl.ds(r0, ROWS_PER_TILE), :])


RP3 = N // NW


@functools.partial(
    pl.kernel,
    out_type=_f32((N, 16)),
    mesh=_MESH,
    scratch_types=[
        pltpu.VMEM((RP3, 16), jnp.float32),
        pltpu.VMEM((RP3, 16), jnp.float32),
    ],
)
def _p3(parts_hbm, s_out, abuf, bbuf):
    cid = lax.axis_index("c")
    sid = lax.axis_index("s")
    r0 = (cid * NS + sid) * RP3
    pltpu.sync_copy(parts_hbm.at[0, pl.ds(r0, RP3), :], abuf)
    pltpu.sync_copy(parts_hbm.at[1, pl.ds(r0, RP3), :], bbuf)

    def _row(i, _):
        abuf[i, :] = abuf[i, :] + bbuf[i, :]
        return 0
    lax.fori_loop(0, RP3, _row, 0)
    pltpu.sync_copy(abuf, s_out.at[pl.ds(r0, RP3)])


EPT4 = E // NS
NCH4 = EPT4 // C
RHALF = ROWS_PER_TILE // 2
HPC = NH // NC


def _make_p4(first_round):
    out_type = [_f32((N, NC, HH))]
    if first_round:
        out_type.append(_f32((NC, HPC, E)))

    @functools.partial(
        pl.kernel,
        out_type=out_type,
        mesh=_MESH,
        scratch_types=[
            pltpu.VMEM((C,), jnp.int32),
            pltpu.VMEM((C,), jnp.int32),
            pltpu.VMEM((C,), jnp.int32),
            pltpu.VMEM((C, HH), jnp.float32),
            pltpu.VMEM((HPC, C), jnp.float32),
            pltpu.VMEM((C, 16), jnp.float32),
            pltpu.VMEM((RHALF, HH), jnp.float32),
            pltpu.VMEM((RHALF, HH), jnp.float32),
            pltpu.VMEM_SHARED((N, HH), jnp.float32),
            pltpu.SemaphoreType.DMA,
            pltpu.SemaphoreType.DMA,
        ],
    )
    def _p4(h2_hbm, v3_hbm, src_hbm, dst_hbm, w1_hbm, *rest):
        if first_round:
            s_hbm = rest[0]
            h_out, attn_out = rest[1], rest[2]
            scr = rest[3:]
        else:
            s_hbm = None
            h_out, attn_out = rest[0], None
            scr = rest[1:]
        (svec, dvec, idxv, rows, abuf, srows, bigbuf, vbuf, acc,
         sem_a, sem_b) = scr

        cid = lax.axis_index("c")
        sid = lax.axis_index("s")
        base0 = sid * EPT4
        zero16 = jnp.zeros((L,), jnp.float32)
        lanes = lax.iota(jnp.int32, L)

        def _zrow(i, _):
            for g in range(GPH):
                bigbuf[i, pl.ds(g * L, L)] = zero16
            return 0
        lax.fori_loop(0, RHALF, _zrow, 0)
        for half in range(2):
            pltpu.sync_copy(
                bigbuf,
                acc.at[pl.ds(sid * ROWS_PER_TILE + half * RHALF, RHALF)])
        plsc.subcore_barrier()

        def _chunk(c, _):
            base = base0 + c * C
            pltpu.sync_copy(src_hbm.at[pl.ds(base, C)], svec)
            pltpu.sync_copy(dst_hbm.at[pl.ds(base, C)], dvec)

            def _mkidx(i, _):
                idxv[pl.ds(i * L, L)] = svec[pl.ds(i * L, L)] * 2 + cid
                return 0
            lax.fori_loop(0, C // L, _mkidx, 0)

            cp_h = pltpu.async_copy(h2_hbm.at[idxv], rows, sem_a)

            if first_round:
                pltpu.sync_copy(
                    w1_hbm.at[pl.ds(cid * HPC, HPC), pl.ds(base, C)], abuf)
                cp_s = pltpu.async_copy(s_hbm.at[dvec], srows, sem_b)
                cp_s.wait()
                for g in range(HPC):
                    hglob = jnp.full((L,), g, jnp.int32) + cid * HPC
                    for grp in range(C // L):
                        ex = abuf[g, pl.ds(grp * L, L)]
                        sv = plsc.load_gather(srows, [grp * L + lanes, hglob])
                        abuf[g, pl.ds(grp * L, L)] = ex / sv
                pltpu.sync_copy(abuf, attn_out.at[cid, :, pl.ds(base, C)])
            else:
                pltpu.sync_copy(w1_hbm.at[cid, :, pl.ds(base, C)], abuf)

            cp_h.wait()

            def _edge(i, _):
                for j in range(4):
                    e = i * 4 + j
                    for g in range(GPH):
                        a = abuf[g, e]
                        rows[e, pl.ds(g * L, L)] = rows[e, pl.ds(g * L, L)] * a
                return 0
            lax.fori_loop(0, C // 4, _edge, 0)

            pltpu.sync_copy(rows, acc.at[dvec], add=True)
            return 0

        lax.fori_loop(0, NCH4, _chunk, 0)
        plsc.subcore_barrier()

        for half in range(2):
            n0 = sid * ROWS_PER_TILE + half * RHALF
            pltpu.sync_copy(acc.at[pl.ds(n0, RHALF)], bigbuf)
            pltpu.sync_copy(v3_hbm.at[pl.ds(n0, RHALF), cid, :], vbuf)

            def _row(i, _):
                for g in range(GPH):
                    sl = pl.ds(g * L, L)
                    bigbuf[i, sl] = ((1.0 - ALPHA) * bigbuf[i, sl]
                                     + ALPHA * vbuf[i, sl])
                return 0
            lax.fori_loop(0, RHALF, _row, 0)
            pltpu.sync_copy(bigbuf, h_out.at[pl.ds(n0, RHALF), cid, :])

    return _p4


_p4_first = _make_p4(True)
_p4_rest = _make_p4(False)


def kernel(hidden_states, attention_mask, edge_index, Wq, bq, Wk, bk, Wv, bv,
           Wo, bo, ln_g, ln_b):
    x = hidden_states.reshape(N, H)
    src = edge_index[0]
    dst = edge_index[1]

    q, k, v = _qkv(x, Wq, Wk, Wv, bq, bk, bv)
    ex_t, parts = _p2(k, q, src, dst)
    s_pad = _p3(parts)

    v2 = v.reshape(NC * N, HH)
    v3 = v.reshape(N, NC, HH)
    h, attn_t = _p4_first(v2, v3, src, dst, ex_t, s_pad)
    for _ in range(4):
        h = _p4_rest(h.reshape(NC * N, HH), v3, src, dst, attn_t)

    y = _out_proj(h.reshape(N, H), x, Wo, bo, ln_g, ln_b)
    return y.reshape(B, S, H)


# speedup vs baseline: 23.4956x; 23.4956x over previous
import functools

import jax
import jax.numpy as jnp
from jax import lax
from jax.experimental import pallas as pl
from jax.experimental.pallas import tpu as pltpu
from jax.experimental.pallas import tpu_sc as plsc

B, S, H, NH = 4, 4096, 128, 8
HD = H // NH
N = B * S
E = 262144
LN_EPS = 1e-5
ALPHA = 0.1

NC, NS, L = 2, 16, 16
NW = NC * NS
C = 128
HH = H // NC
GPH = HH // L
HPC = NH // NC
NCHG = E // C

EPT2 = E // NW
NCH2 = EPT2 // C
EPT4 = E // NS
NCH4 = EPT4 // C
ROWS_PER_TILE = N // NS
RZ = 256
RSUB = 128

_MESH = plsc.VectorSubcoreMesh(core_axis_name="c", subcore_axis_name="s")
_SC_PARAMS = pltpu.CompilerParams(needs_layout_passes=False,
                                  use_tc_tiling_on_sc=False)


def _f32(shape):
    return jax.ShapeDtypeStruct(shape, jnp.float32)


def _qkv_body(x_ref, wq_ref, wk_ref, wv_ref, bq_ref, bk_ref, bv_ref,
              q_ref, k_ref, vt_ref, vts_ref):
    x = x_ref[...]
    dg = lambda a, w: lax.dot_general(a, w, (((1,), (1,)), ((), ())),
                                      preferred_element_type=jnp.float32)
    q_ref[...] = (dg(x, wq_ref[...]) + bq_ref[...]) * (1.0 / 4.0)
    k_ref[...] = dg(x, wk_ref[...]) + bk_ref[...]
    v = dg(x, wv_ref[...]) + bv_ref[...]
    vt_ref[0] = v[:, :HH]
    vt_ref[1] = v[:, HH:]
    vts_ref[0] = v[:, :HH] * (ALPHA / (1.0 - ALPHA))
    vts_ref[1] = v[:, HH:] * (ALPHA / (1.0 - ALPHA))


def _qkv(x, Wq, Wk, Wv, bq, bk, bv):
    blk = 1024
    row_spec = pl.BlockSpec((blk, H), lambda i: (i, 0))
    w_spec = pl.BlockSpec((H, H), lambda i: (0, 0))
    b_spec = pl.BlockSpec((1, H), lambda i: (0, 0))
    vt_spec = pl.BlockSpec((NC, blk, HH), lambda i: (0, i, 0))
    return pl.pallas_call(
        _qkv_body,
        grid=(N // blk,),
        in_specs=[row_spec, w_spec, w_spec, w_spec, b_spec, b_spec, b_spec],
        out_specs=[row_spec, row_spec, vt_spec, vt_spec],
        out_shape=[_f32((N, H))] * 2 + [_f32((NC, N, HH))] * 2,
    )(x, Wq, Wk, Wv, bq.reshape(1, H), bk.reshape(1, H), bv.reshape(1, H))


def _out_body(h0_ref, h1_ref, x_ref, wo_ref, bo_ref, g_ref, b_ref, y_ref):
    h = jnp.concatenate([h0_ref[...], h1_ref[...]], axis=1)
    y = lax.dot_general(h, wo_ref[...], (((1,), (1,)), ((), ())),
                        preferred_element_type=jnp.float32)
    y = y + bo_ref[...] + x_ref[...]
    mu = jnp.mean(y, axis=-1, keepdims=True)
    var = jnp.mean((y - mu) ** 2, axis=-1, keepdims=True)
    y_ref[...] = (y - mu) * lax.rsqrt(var + LN_EPS) * g_ref[...] + b_ref[...]


def _out_proj(h2, x, Wo, bo, g, b):
    blk = 1024
    row_spec = pl.BlockSpec((blk, H), lambda i: (i, 0))
    half_spec = pl.BlockSpec((blk, HH), lambda i: (i, 0))
    w_spec = pl.BlockSpec((H, H), lambda i: (0, 0))
    b_spec = pl.BlockSpec((1, H), lambda i: (0, 0))
    return pl.pallas_call(
        _out_body,
        grid=(N // blk,),
        in_specs=[half_spec, half_spec, row_spec, w_spec, b_spec, b_spec,
                  b_spec],
        out_specs=row_spec,
        out_shape=_f32((N, H)),
    )(h2[:N], h2[N:], x, Wo, bo.reshape(1, H), g.reshape(1, H),
      b.reshape(1, H))


@functools.partial(
    pl.kernel,
    out_type=[_f32((NCHG, NH, C)), _f32((NC, N, 16))],
    mesh=_MESH,
    compiler_params=_SC_PARAMS,
    scratch_types=[
        pltpu.VMEM((NCH2, C), jnp.int32),
        pltpu.VMEM((NCH2, C), jnp.int32),
        pltpu.VMEM((2, C, H), jnp.float32),
        pltpu.VMEM((2, C, H), jnp.float32),
        pltpu.VMEM((2, NH, C), jnp.float32),
        pltpu.VMEM((2, C, 16), jnp.float32),
        pltpu.VMEM((RZ, 16), jnp.float32),
        pltpu.VMEM_SHARED((N, 16), jnp.float32),
        pltpu.SemaphoreType.DMA, pltpu.SemaphoreType.DMA,
        pltpu.SemaphoreType.DMA, pltpu.SemaphoreType.DMA,
        pltpu.SemaphoreType.DMA, pltpu.SemaphoreType.DMA,
    ],
)
def _p2(k_hbm, q_hbm, src_hbm, dst_hbm, ex_out, parts_out,
        src_all, dst_all, krows, qrows, sct, pad, zbuf, denom,
        gk0, gk1, gq0, gq1, se0, se1):
    cid = lax.axis_index("c")
    sid = lax.axis_index("s")
    wid = cid * NS + sid
    zero16 = jnp.zeros((L,), jnp.float32)
    lanes = lax.iota(jnp.int32, L)
    ksem = (gk0, gk1)
    qsem = (gq0, gq1)
    esem = (se0, se1)

    pltpu.sync_copy(src_hbm.at[wid], src_all)
    pltpu.sync_copy(dst_hbm.at[wid], dst_all)

    def _zrow(i, _):
        zbuf[i, :] = zero16
        return 0
    lax.fori_loop(0, RZ, _zrow, 0)
    for p in range(2):
        def _prow(i, _):
            pad[p, i, :] = zero16
            return 0
        lax.fori_loop(0, C, _prow, 0)
    for t in range(ROWS_PER_TILE // RZ):
        pltpu.sync_copy(
            zbuf, denom.at[pl.ds(sid * ROWS_PER_TILE + t * RZ, RZ)])
    plsc.subcore_barrier()

    def _issue(c, p):
        pltpu.async_copy(k_hbm.at[src_all.at[c]], krows.at[p], ksem[p])
        pltpu.async_copy(q_hbm.at[dst_all.at[c]], qrows.at[p], qsem[p])

    _issue(0, 0)

    def _phase(c, p):
        _issue(lax.rem(c + 1, NCH2), 1 - p)
        @pl.when(c >= 2)
        def _():
            pltpu.make_async_copy(sct.at[p], ex_out.at[0], esem[p]).wait()
        pltpu.make_async_copy(k_hbm.at[src_all.at[0]], krows.at[p],
                              ksem[p]).wait()
        pltpu.make_async_copy(q_hbm.at[dst_all.at[0]], qrows.at[p],
                              qsem[p]).wait()

        def _grp(grp, _):
            e0 = grp * L
            erows = e0 + lanes
            for h in range(NH):
                acc = jnp.zeros((L,), jnp.float32)
                for j in range(HD):
                    col = jnp.full((L,), h * HD + j, jnp.int32)
                    kv = plsc.load_gather(krows.at[p], [erows, col])
                    qv = plsc.load_gather(qrows.at[p], [erows, col])
                    acc = acc + kv * qv
                ex = jnp.exp(acc)
                sct[p, h, pl.ds(e0, L)] = ex
                plsc.store_scatter(
                    pad.at[p], [erows, jnp.full((L,), h, jnp.int32)], ex)
            return 0
        lax.fori_loop(0, C // L, _grp, 0)

        pltpu.async_copy(sct.at[p], ex_out.at[wid * NCH2 + c], esem[p])
        pltpu.sync_copy(pad.at[p], denom.at[dst_all.at[c]], add=True)

    def _pair(c2, _):
        _phase(c2 * 2, 0)
        _phase(c2 * 2 + 1, 1)
        return 0
    lax.fori_loop(0, NCH2 // 2, _pair, 0)

    pltpu.make_async_copy(k_hbm.at[src_all.at[0]], krows.at[0], gk0).wait()
    pltpu.make_async_copy(q_hbm.at[src_all.at[0]], qrows.at[0], gq0).wait()
    for p in range(2):
        pltpu.make_async_copy(sct.at[p], ex_out.at[0], esem[p]).wait()
    plsc.subcore_barrier()

    for t in range(ROWS_PER_TILE // RZ):
        r0 = sid * ROWS_PER_TILE + t * RZ
        pltpu.sync_copy(denom.at[pl.ds(r0, RZ)], zbuf)
        pltpu.sync_copy(zbuf, parts_out.at[cid, pl.ds(r0, RZ), :])


@functools.partial(
    pl.kernel,
    out_type=[_f32((NC, NCHG, HPC, C)), _f32((NC, N, 16))],
    mesh=_MESH,
    compiler_params=_SC_PARAMS,
    scratch_types=[
        pltpu.VMEM((NCH4, C), jnp.int32),
        pltpu.VMEM((RZ, 16), jnp.float32),
        pltpu.VMEM((RZ, 16), jnp.float32),
        pltpu.VMEM((2, C, 16), jnp.float32),
        pltpu.VMEM((2, HPC, C), jnp.float32),
        pltpu.VMEM((2, HPC, C), jnp.float32),
        pltpu.SemaphoreType.DMA, pltpu.SemaphoreType.DMA,
        pltpu.SemaphoreType.DMA, pltpu.SemaphoreType.DMA,
        pltpu.SemaphoreType.DMA, pltpu.SemaphoreType.DMA,
    ],
)
def _p3(parts_hbm, ex_hbm, dst_hbm, attn_out, s_out,
        dst_all, abuf, bbuf, srows, ebuf, abt,
        gs0, gs1, ge0, ge1, sa0, sa1):
    cid = lax.axis_index("c")
    sid = lax.axis_index("s")
    lanes = lax.iota(jnp.int32, L)
    ssem = (gs0, gs1)
    esem = (ge0, ge1)
    asem = (sa0, sa1)

    pltpu.sync_copy(dst_hbm.at[sid], dst_all)
    for t in range(ROWS_PER_TILE // RZ):
        r0 = sid * ROWS_PER_TILE + t * RZ
        pltpu.sync_copy(parts_hbm.at[0, pl.ds(r0, RZ), :], abuf)
        pltpu.sync_copy(parts_hbm.at[1, pl.ds(r0, RZ), :], bbuf)

        def _row(i, _):
            abuf[i, :] = abuf[i, :] + bbuf[i, :]
            return 0
        lax.fori_loop(0, RZ, _row, 0)
        pltpu.sync_copy(abuf, s_out.at[cid, pl.ds(r0, RZ), :])
    plsc.subcore_barrier()

    def _issue(c, p):
        pltpu.async_copy(s_out.at[cid].at[dst_all.at[c]], srows.at[p],
                         ssem[p])
        pltpu.async_copy(ex_hbm.at[sid * NCH4 + c, pl.ds(cid * HPC, HPC), :],
                         ebuf.at[p], esem[p])

    _issue(0, 0)

    def _phase(c, p):
        _issue(lax.rem(c + 1, NCH4), 1 - p)
        @pl.when(c >= 2)
        def _():
            pltpu.make_async_copy(abt.at[p], attn_out.at[0, 0],
                                  asem[p]).wait()
        pltpu.make_async_copy(parts_hbm.at[0, pl.ds(0, C), :], srows.at[p],
                              ssem[p]).wait()
        pltpu.make_async_copy(ex_hbm.at[0, pl.ds(cid * HPC, HPC), :],
                              ebuf.at[p], esem[p]).wait()

        def _grp(grp, _):
            e0 = grp * L
            erows = e0 + lanes
            for g in range(HPC):
                hglob = jnp.full((L,), g, jnp.int32) + cid * HPC
                ex = ebuf[p, g, pl.ds(e0, L)]
                sv = plsc.load_gather(srows.at[p], [erows, hglob])
                abt[p, g, pl.ds(e0, L)] = ex / sv
            return 0
        lax.fori_loop(0, C // L, _grp, 0)
        pltpu.async_copy(abt.at[p], attn_out.at[cid, sid * NCH4 + c],
                         asem[p])

    def _pair(c2, _):
        _phase(c2 * 2, 0)
        _phase(c2 * 2 + 1, 1)
        return 0
    lax.fori_loop(0, NCH4 // 2, _pair, 0)

    pltpu.make_async_copy(parts_hbm.at[0, pl.ds(0, C), :], srows.at[0],
                          gs0).wait()
    pltpu.make_async_copy(ex_hbm.at[0, pl.ds(cid * HPC, HPC), :], ebuf.at[0],
                          ge0).wait()
    for p in range(2):
        pltpu.make_async_copy(abt.at[p], attn_out.at[0, 0], asem[p]).wait()


@functools.partial(
    pl.kernel,
    out_type=[_f32((NC * N, HH)), _f32((NC * N, HH)), _f32((NC * N, HH))],
    mesh=_MESH,
    compiler_params=_SC_PARAMS,
    scratch_types=[
        pltpu.VMEM((NCH4, C), jnp.int32),
        pltpu.VMEM((NCH4, C), jnp.int32),
        pltpu.VMEM((2, C, HH), jnp.float32),
        pltpu.VMEM((2, HPC, C), jnp.float32),
        pltpu.VMEM((RSUB, HH), jnp.float32),
        pltpu.VMEM_SHARED((N, HH), jnp.float32),
        pltpu.SemaphoreType.DMA, pltpu.SemaphoreType.DMA,
        pltpu.SemaphoreType.DMA, pltpu.SemaphoreType.DMA,
    ],
)
def _p4(vt_hbm, vts_hbm, attn_hbm, src_hbm, dst_hbm, ha_hbm, hb_hbm, hf_hbm,
        src_all, dst_all, rows, abuf, bigbuf, acc,
        g0, g1, a0, a1):
    cid = lax.axis_index("c")
    sid = lax.axis_index("s")
    lanes = lax.iota(jnp.int32, L)
    gsem = (g0, g1)
    asem = (a0, a1)
    off = cid * N
    tile0 = sid * ROWS_PER_TILE

    pltpu.sync_copy(src_hbm.at[sid], src_all)
    pltpu.sync_copy(dst_hbm.at[sid], dst_all)

    def _off(i, _):
        for j in range(C // L):
            sl = pl.ds(j * L, L)
            src_all[i, sl] = src_all[i, sl] + off
        return 0
    lax.fori_loop(0, NCH4, _off, 0)

    def _init_acc():
        for t in range(ROWS_PER_TILE // RSUB):
            n0 = tile0 + t * RSUB
            pltpu.sync_copy(vts_hbm.at[pl.ds(off + n0, RSUB)], bigbuf)
            pltpu.sync_copy(bigbuf, acc.at[pl.ds(n0, RSUB)])

    _init_acc()
    plsc.subcore_barrier()

    def _round(tbl_hbm, out_hbm, scale, last):
        def _issue(c, p):
            pltpu.async_copy(tbl_hbm.at[src_all.at[c]], rows.at[p], gsem[p])
            pltpu.async_copy(attn_hbm.at[cid, sid * NCH4 + c], abuf.at[p],
                             asem[p])

        _issue(0, 0)

        def _phase(c, p):
            _issue(lax.rem(c + 1, NCH4), 1 - p)
            pltpu.make_async_copy(tbl_hbm.at[src_all.at[0]], rows.at[p],
                                  gsem[p]).wait()
            pltpu.make_async_copy(attn_hbm.at[cid, 0], abuf.at[p],
                                  asem[p]).wait()

            def _egrp(t, _):
                e0 = t * L
                erows = e0 + lanes
                for g in range(GPH):
                    a_vec = abuf[p, g, pl.ds(e0, L)] * scale
                    for j in range(L):
                        col = jnp.full((L,), g * L + j, jnp.int32)
                        hv = plsc.load_gather(rows.at[p], [erows, col])
                        plsc.store_scatter(rows.at[p], [erows, col],
                                           hv * a_vec)
                return 0
            lax.fori_loop(0, C // L, _egrp, 0)

            pltpu.sync_copy(rows.at[p], acc.at[dst_all.at[c]], add=True)

        def _pair(c2, _):
            _phase(c2 * 2, 0)
            _phase(c2 * 2 + 1, 1)
            return 0
        lax.fori_loop(0, NCH4 // 2, _pair, 0)

        pltpu.make_async_copy(tbl_hbm.at[src_all.at[0]], rows.at[0],
                              gsem[0]).wait()
        pltpu.make_async_copy(attn_hbm.at[cid, 0], abuf.at[0],
                              asem[0]).wait()
        plsc.subcore_barrier()

        if not last:
            for t in range(ROWS_PER_TILE // RSUB):
                n0 = tile0 + t * RSUB
                pltpu.sync_copy(acc.at[pl.ds(n0, RSUB)], bigbuf)
                pltpu.sync_copy(bigbuf, out_hbm.at[pl.ds(off + n0, RSUB)])
            _init_acc()
            plsc.subcore_barrier()
        else:
            for t in range(ROWS_PER_TILE // RSUB):
                n0 = tile0 + t * RSUB
                pltpu.sync_copy(acc.at[pl.ds(n0, RSUB)], bigbuf)

                def _row(i, _):
                    for g in range(GPH):
                        sl = pl.ds(g * L, L)
                        bigbuf[i, sl] = (1.0 - ALPHA) * bigbuf[i, sl]
                    return 0
                lax.fori_loop(0, RSUB, _row, 0)
                pltpu.sync_copy(bigbuf, out_hbm.at[pl.ds(off + n0, RSUB)])

    _round(vt_hbm, ha_hbm, 1.0, False)
    _round(ha_hbm, hb_hbm, 1.0 - ALPHA, False)
    _round(hb_hbm, ha_hbm, 1.0 - ALPHA, False)
    _round(ha_hbm, hb_hbm, 1.0 - ALPHA, False)
    _round(hb_hbm, hf_hbm, 1.0 - ALPHA, True)


def kernel(hidden_states, attention_mask, edge_index, Wq, bq, Wk, bk, Wv, bv,
           Wo, bo, ln_g, ln_b):
    x = hidden_states.reshape(N, H)
    src = edge_index[0]
    dst = edge_index[1]
    src2 = src.reshape(NW, NCH2, C)
    dst2 = dst.reshape(NW, NCH2, C)
    src4 = src.reshape(NS, NCH4, C)
    dst4 = dst.reshape(NS, NCH4, C)

    q, k, vt, vts = _qkv(x, Wq, Wk, Wv, bq, bk, bv)
    ex_t, parts = _p2(k, q, src2, dst2)
    attn_t, _ = _p3(parts, ex_t, dst4)
    vt2 = vt.reshape(NC * N, HH)
    vts2 = vts.reshape(NC * N, HH)
    h, _, _ = _p4(vt2, vts2, attn_t, src4, dst4)

    y = _out_proj(h, x, Wo, bo, ln_g, ln_b)
    return y.reshape(B, S, H)
